# Initial kernel scaffold; baseline (speedup 1.0000x reference)
import jax
import jax.numpy as jnp
from jax.experimental import pallas as pl


def kernel(hidden_states, router_weight, router_bias, gate_up_proj, gate_up_proj_bias, down_proj, down_proj_bias):
    raise NotImplementedError("write your pallas kernel here")


# speedup vs baseline: 1.6180x; 1.6180x over previous
import jax
import jax.numpy as jnp
from jax.experimental import pallas as pl
from jax.experimental.pallas import tpu as pltpu

_B, _S, _H = 1, 2048, 768
_E, _K, _I = 8, 2, 1536
_ALPHA = 1.702
_LIMIT = 7.0
_T = _B * _S
_NT = 2
_TH = _T // _NT
_SUB = 512


def _moe_kernel(x_ref, rw_ref, rb_ref, gup_ref, gub_ref, dwn_ref, dnb_ref,
                out_ref, scores_ref, sc_scratch):
    e = pl.program_id(1)

    @pl.when(e == 0)
    def _():
        xb = x_ref[...].astype(jnp.bfloat16)
        logits = jnp.dot(xb, rw_ref[...].astype(jnp.bfloat16),
                         preferred_element_type=jnp.float32) + rb_ref[...]
        iota = jax.lax.broadcasted_iota(jnp.int32, (_TH, _E), 1)
        m0 = jnp.max(logits, axis=1, keepdims=True)
        i0 = jnp.min(jnp.where(logits == m0, iota, _E), axis=1, keepdims=True)
        oh0 = iota == i0
        l1 = jnp.where(oh0, -jnp.inf, logits)
        m1 = jnp.max(l1, axis=1, keepdims=True)
        i1 = jnp.min(jnp.where(l1 == m1, iota, _E), axis=1, keepdims=True)
        oh1 = iota == i1
        w0 = jax.nn.sigmoid(m0 - m1)
        w1 = jax.nn.sigmoid(m1 - m0)
        scores = w0 * oh0.astype(jnp.float32) + w1 * oh1.astype(jnp.float32)
        scores_ref[...] = scores
        sc_scratch[...] = scores
        out_ref[...] = jnp.zeros_like(out_ref)

    wg = gup_ref[0]
    wd = dwn_ref[0]
    bg = gub_ref[0]
    bd = dnb_ref[0]
    for j in range(_TH // _SUB):
        rows = pl.ds(j * _SUB, _SUB)
        xs = x_ref[rows, :].astype(jnp.bfloat16)
        gu = jnp.dot(xs, wg, preferred_element_type=jnp.float32) + bg
        gu = jnp.minimum(gu, _LIMIT)
        gate = gu[:, :_I]
        up = jnp.maximum(gu[:, _I:], -_LIMIT)
        glu = gate * jax.nn.sigmoid(gate * _ALPHA)
        act = (up + 1.0) * glu
        y = jnp.dot(act.astype(jnp.bfloat16), wd,
                    preferred_element_type=jnp.float32) + bd
        tile = sc_scratch[rows, :]
        iota_sub = jax.lax.broadcasted_iota(jnp.int32, (_SUB, _E), 1)
        sc = jnp.sum(tile * (iota_sub == e).astype(jnp.float32),
                     axis=1, keepdims=True)
        out_ref[rows, :] += sc * y


def kernel(hidden_states, router_weight, router_bias, gate_up_proj,
           gate_up_proj_bias, down_proj, down_proj_bias):
    x = hidden_states.reshape(_T, _H)
    rb = router_bias.reshape(1, _E)
    gup = gate_up_proj.astype(jnp.bfloat16)
    dwn = down_proj.astype(jnp.bfloat16)
    gub = gate_up_proj_bias.reshape(_E, 1, 2 * _I)
    dnb = down_proj_bias.reshape(_E, 1, _H)

    out, scores = pl.pallas_call(
        _moe_kernel,
        grid=(_NT, _E),
        in_specs=[
            pl.BlockSpec((_TH, _H), lambda i, e: (i, 0)),
            pl.BlockSpec((_H, _E), lambda i, e: (0, 0)),
            pl.BlockSpec((1, _E), lambda i, e: (0, 0)),
            pl.BlockSpec((1, _H, 2 * _I), lambda i, e: (e, 0, 0)),
            pl.BlockSpec((1, 1, 2 * _I), lambda i, e: (e, 0, 0)),
            pl.BlockSpec((1, _I, _H), lambda i, e: (e, 0, 0)),
            pl.BlockSpec((1, 1, _H), lambda i, e: (e, 0, 0)),
        ],
        out_specs=[
            pl.BlockSpec((_TH, _H), lambda i, e: (i, 0)),
            pl.BlockSpec((_TH, _E), lambda i, e: (i, 0)),
        ],
        out_shape=[
            jax.ShapeDtypeStruct((_T, _H), jnp.float32),
            jax.ShapeDtypeStruct((_T, _E), jnp.float32),
        ],
        scratch_shapes=[pltpu.VMEM((_TH, _E), jnp.float32)],
        compiler_params=pltpu.CompilerParams(
            dimension_semantics=("parallel", "arbitrary"),
        ),
    )(x, router_weight, rb, gup, gub, dwn, dnb)

    return out.reshape(_B, _S, _H), scores
